# baseline (speedup 1.0000x reference)
import functools

import jax
import jax.numpy as jnp
from jax import lax
from jax.experimental import pallas as pl
from jax.experimental.pallas import tpu as pltpu
from jax.experimental.pallas import tpu_sc as plsc

VOCAB = 1000000
EMB = 64
BATCH = 16384
NEG = 10

NC = 2
NS = 16
NW = NC * NS
VSLICE = 31248
SLAB = VSLICE + 64
SLAB_SC = NS * VSLICE + 64
NITEMS = BATCH * (1 + NEG)
BANK = 6400
SCAN_CH = 4096


def _sc_scores_body(pos_hbm, neg_hbm, vt_hbm, out_hbm,
                    score_out, tag_out,
                    sidx, l_woff, l_b, l_sc, acc,
                    slab0, vd0, sh_slab, sh_vd, sem0):
    cid = lax.axis_index("c")
    sid = lax.axis_index("s")
    wid = cid * NS + sid
    iota16 = lax.broadcasted_iota(jnp.int32, (16,), 0)
    lo = wid * VSLICE
    lo_sc = cid * (NS * VSLICE)
    limit = jnp.where(wid == NW - 1, SLAB, VSLICE)

    def init(g, _):
        sl = pl.ds(g * 16, 16)
        l_woff[sl] = jnp.zeros((16,), jnp.int32)
        l_b[sl] = jnp.zeros((16,), jnp.int32)
        l_sc[sl] = jnp.full((16,), -1, jnp.int32)
        acc[sl] = jnp.zeros((16,), jnp.float32)
        return 0
    lax.fori_loop(0, BANK // 16, init, 0, unroll=8)

    def scan_chunk(idx_hbm, chunk, is_neg, cnt):
        pltpu.sync_copy(idx_hbm.at[pl.ds(chunk * SCAN_CH, SCAN_CH)], sidx)

        def body(i, cnt):
            x = sidx[pl.ds(i * 16, 16)]
            kv = chunk * SCAN_CH + i * 16 + iota16
            w = x - lo
            m = (w >= 0) & (w < limit)
            b = kv // NEG if is_neg else kv
            sc = BATCH + kv if is_neg else kv
            plsc.store_compressed(l_woff.at[pl.ds(cnt, 16)], w, mask=m)
            plsc.store_compressed(l_b.at[pl.ds(cnt, 16)], b, mask=m)
            plsc.store_compressed(l_sc.at[pl.ds(cnt, 16)], sc, mask=m)
            return cnt + plsc.all_reduce_population_count(m)[0]
        return lax.fori_loop(0, SCAN_CH // 16, body, cnt, unroll=4)

    cnt = 0
    for c in range(BATCH // SCAN_CH):
        cnt = scan_chunk(pos_hbm, c, False, cnt)
    for c in range(BATCH * NEG // SCAN_CH):
        cnt = scan_chunk(neg_hbm, c, True, cnt)

    def dsel(t):
        return lax.rem(t + cid * (EMB // 2), EMB)

    def stage(d):
        pltpu.async_copy(out_hbm.at[pl.ds(d * VOCAB + lo_sc, SLAB_SC)],
                         sh_slab, sem0)
        pltpu.async_copy(vt_hbm.at[pl.ds(d * BATCH, BATCH)], sh_vd, sem0)

    def drain():
        pltpu.make_async_copy(out_hbm.at[pl.ds(0, SLAB_SC)],
                              sh_slab, sem0).wait()
        pltpu.make_async_copy(out_hbm.at[pl.ds(0, BATCH)],
                              sh_vd, sem0).wait()

    def pull():
        pltpu.sync_copy(sh_slab.at[pl.ds(sid * VSLICE, SLAB)], slab0)
        pltpu.sync_copy(sh_vd, vd0)

    def compute():
        def body(i, _):
            sl = pl.ds(i * 16, 16)
            u = plsc.load_gather(slab0, [l_woff[sl]])
            v = plsc.load_gather(vd0, [l_b[sl]])
            acc[sl] = acc[sl] + u * v
            return 0
        lax.fori_loop(0, BANK // 16, body, 0)

    @pl.when(sid == 0)
    def _():
        stage(dsel(0))

    def step(t, _):
        @pl.when(sid == 0)
        def _():
            drain()
        plsc.subcore_barrier()
        pull()
        plsc.subcore_barrier()

        @pl.when((sid == 0) & (t < EMB - 1))
        def _():
            stage(dsel(t + 1))
        compute()
        return 0
    lax.fori_loop(0, EMB, step, 0)

    pltpu.sync_copy(acc, score_out.at[pl.ds(wid * BANK, BANK)])
    pltpu.sync_copy(l_sc, tag_out.at[pl.ds(wid * BANK, BANK)])


def _loss_body(s_ref, t_ref, out_ref):
    s = s_ref[...]
    t = t_ref[...]
    valid = t >= 0
    is_pos = t < BATCH
    x = jnp.where(is_pos, s, -s)
    ls = jnp.minimum(x, 0.0) - jnp.log1p(jnp.exp(-jnp.abs(x)))
    w = jnp.where(is_pos, 1.0 / BATCH, 1.0 / (BATCH * NEG))
    out_ref[0, 0] = -jnp.sum(jnp.where(valid, ls * w, 0.0))


@jax.jit
def kernel(in_embed, out_embed, center, pos, neg):
    center = center.astype(jnp.int32)
    pos = pos.astype(jnp.int32)
    neg_flat = jnp.reshape(neg.astype(jnp.int32), (BATCH * NEG,))
    v = jnp.take(in_embed, center, axis=0)
    vt_flat = jnp.reshape(v.T, (EMB * BATCH,))
    out_flat = jnp.reshape(out_embed.T, (VOCAB * EMB,))

    mesh = plsc.VectorSubcoreMesh(core_axis_name="c", subcore_axis_name="s")
    sc_scores = functools.partial(
        pl.kernel,
        mesh=mesh,
        compiler_params=pltpu.CompilerParams(
            needs_layout_passes=False, use_tc_tiling_on_sc=False),
        out_type=[jax.ShapeDtypeStruct((NW * BANK,), jnp.float32),
                  jax.ShapeDtypeStruct((NW * BANK,), jnp.int32)],
        scratch_types=[
            pltpu.VMEM((SCAN_CH,), jnp.int32),
            pltpu.VMEM((BANK,), jnp.int32),
            pltpu.VMEM((BANK,), jnp.int32),
            pltpu.VMEM((BANK,), jnp.int32),
            pltpu.VMEM((BANK,), jnp.float32),
            pltpu.VMEM((SLAB,), jnp.float32),
            pltpu.VMEM((BATCH,), jnp.float32),
            pltpu.VMEM_SHARED((SLAB_SC,), jnp.float32),
            pltpu.VMEM_SHARED((BATCH,), jnp.float32),
            pltpu.SemaphoreType.DMA,
        ],
    )(_sc_scores_body)
    scores, tags = sc_scores(pos, neg_flat, vt_flat, out_flat)

    loss = pl.pallas_call(
        _loss_body,
        out_shape=jax.ShapeDtypeStruct((1, 1), jnp.float32),
        out_specs=pl.BlockSpec(memory_space=pltpu.SMEM),
    )(jnp.reshape(scores, (NW * BANK // 128, 128)),
      jnp.reshape(tags, (NW * BANK // 128, 128)))
    return loss[0, 0]


# speedup vs baseline: 4.3199x; 4.3199x over previous
import functools
import math

import jax
import jax.numpy as jnp
from jax import lax
from jax.experimental import pallas as pl
from jax.experimental.pallas import tpu as pltpu
from jax.experimental.pallas import tpu_sc as plsc

VOCAB = 1000000
EMB = 64
BATCH = 16384
NEG = 10

NC = 2
NS = 16
NW = NC * NS
ROWS_W = BATCH // NW
CHUNK = 64
NCHUNK = ROWS_W // CHUNK
NEG_CH = CHUNK * NEG
NEG_GATHERS = NEG_CH // 128


def _sc_scores_body(center_hbm, pos_hbm, neg_hbm, in_hbm, out_hbm,
                    sp_out, sn_out,
                    idx_c0, idx_c1, idx_p0, idx_p1, idx_n0, idx_n1,
                    rows_v0, rows_v1, rows_p0, rows_p1, rows_n0, rows_n1,
                    sp, sn, sem0, sem1):
    wid = lax.axis_index("s") * NC + lax.axis_index("c")
    idx_c = (idx_c0, idx_c1)
    idx_p = (idx_p0, idx_p1)
    idx_n = (idx_n0, idx_n1)
    rows_v = (rows_v0, rows_v1)
    rows_p = (rows_p0, rows_p1)
    rows_n = (rows_n0, rows_n1)
    sems = (sem0, sem1)
    iota16 = lax.broadcasted_iota(jnp.int32, (16,), 0)

    def issue(c, s):
        base = wid * ROWS_W + c * CHUNK
        pltpu.sync_copy(center_hbm.at[pl.ds(base, CHUNK)], idx_c[s])
        pltpu.sync_copy(pos_hbm.at[pl.ds(base, CHUNK)], idx_p[s])
        pltpu.sync_copy(neg_hbm.at[pl.ds(base * NEG, NEG_CH)], idx_n[s])
        cps = [pltpu.async_copy(in_hbm.at[idx_c[s]], rows_v[s], sems[s]),
               pltpu.async_copy(out_hbm.at[idx_p[s]], rows_p[s], sems[s])]
        for k in range(NEG_GATHERS):
            cps.append(pltpu.async_copy(out_hbm.at[idx_n[s].at[pl.ds(k * 128, 128)]],
                                        rows_n[s].at[pl.ds(k * 128, 128)],
                                        sems[s]))
        return cps

    def compute(c, s):
        for g in range(CHUNK // 16):
            r_idx = g * 16 + iota16
            p_idx = [(g * 16 + iota16) * NEG + j for j in range(NEG)]
            zeros = jnp.zeros((16,), jnp.float32)

            def body(d, accs):
                col = jnp.broadcast_to(d, (16,))
                vv = plsc.load_gather(rows_v[s], [r_idx, col])
                up = plsc.load_gather(rows_p[s], [r_idx, col])
                new = [accs[0] + vv * up]
                for j in range(NEG):
                    un = plsc.load_gather(rows_n[s], [p_idx[j], col])
                    new.append(accs[1 + j] + vv * un)
                return tuple(new)

            accs = lax.fori_loop(0, EMB, body, (zeros,) * (1 + NEG))
            sp[pl.ds(c * CHUNK + g * 16, 16)] = accs[0]
            for j in range(NEG):
                plsc.store_scatter(
                    sn, [(c * CHUNK + g * 16 + iota16) * NEG + j], accs[1 + j])

    cps = issue(0, 0)
    for c in range(NCHUNK):
        s = c % 2
        nxt = issue(c + 1, 1 - s) if c + 1 < NCHUNK else None
        for cp in cps:
            cp.wait()
        compute(c, s)
        cps = nxt

    pltpu.sync_copy(sp, sp_out.at[pl.ds(wid * ROWS_W, ROWS_W)])
    pltpu.sync_copy(sn, sn_out.at[pl.ds(wid * ROWS_W * NEG, ROWS_W * NEG)])


def _loss_body(sp_ref, sn_ref, out_ref):
    ps = sp_ref[...]
    ns = sn_ref[...]
    pls = jnp.minimum(ps, 0.0) - jnp.log1p(jnp.exp(-jnp.abs(ps)))
    nls = jnp.minimum(-ns, 0.0) - jnp.log1p(jnp.exp(-jnp.abs(ns)))
    out_ref[0, 0] = -(jnp.sum(pls) / BATCH) - (jnp.sum(nls) / (BATCH * NEG))


@jax.jit
def kernel(in_embed, out_embed, center, pos, neg):
    center = center.astype(jnp.int32)
    pos = pos.astype(jnp.int32)
    neg_flat = jnp.reshape(neg.astype(jnp.int32), (BATCH * NEG,))

    mesh = plsc.VectorSubcoreMesh(core_axis_name="c", subcore_axis_name="s")
    sc_scores = functools.partial(
        pl.kernel,
        mesh=mesh,
        compiler_params=pltpu.CompilerParams(
            needs_layout_passes=False, use_tc_tiling_on_sc=False),
        out_type=[jax.ShapeDtypeStruct((BATCH,), jnp.float32),
                  jax.ShapeDtypeStruct((BATCH * NEG,), jnp.float32)],
        scratch_types=[
            pltpu.VMEM((CHUNK,), jnp.int32), pltpu.VMEM((CHUNK,), jnp.int32),
            pltpu.VMEM((CHUNK,), jnp.int32), pltpu.VMEM((CHUNK,), jnp.int32),
            pltpu.VMEM((NEG_CH,), jnp.int32),
            pltpu.VMEM((NEG_CH,), jnp.int32),
            pltpu.VMEM((CHUNK, EMB), jnp.float32),
            pltpu.VMEM((CHUNK, EMB), jnp.float32),
            pltpu.VMEM((CHUNK, EMB), jnp.float32),
            pltpu.VMEM((CHUNK, EMB), jnp.float32),
            pltpu.VMEM((NEG_CH, EMB), jnp.float32),
            pltpu.VMEM((NEG_CH, EMB), jnp.float32),
            pltpu.VMEM((ROWS_W,), jnp.float32),
            pltpu.VMEM((ROWS_W * NEG,), jnp.float32),
            pltpu.SemaphoreType.DMA,
            pltpu.SemaphoreType.DMA,
        ],
    )(_sc_scores_body)
    sp, sn = sc_scores(center, pos, neg_flat, in_embed, out_embed)

    loss = pl.pallas_call(
        _loss_body,
        out_shape=jax.ShapeDtypeStruct((1, 1), jnp.float32),
        out_specs=pl.BlockSpec(memory_space=pltpu.SMEM),
    )(jnp.reshape(sp, (BATCH // 128, 128)),
      jnp.reshape(sn, (BATCH * NEG // 128, 128)))
    return loss[0, 0]


# speedup vs baseline: 5.3087x; 1.2289x over previous
import functools
import math

import jax
import jax.numpy as jnp
from jax import lax
from jax.experimental import pallas as pl
from jax.experimental.pallas import tpu as pltpu
from jax.experimental.pallas import tpu_sc as plsc

VOCAB = 1000000
EMB = 64
BATCH = 16384
NEG = 10

NC = 2
NS = 16
NW = NC * NS
ROWS_W = BATCH // NW
CHUNK = 64
NCHUNK = ROWS_W // CHUNK
NEG_CH = CHUNK * NEG
NEG_GATHERS = NEG_CH // 128


def _sc_scores_body(pos_hbm, neg_hbm, v_hbm, out_hbm,
                    sp_out, sn_out,
                    idx_c0, idx_c1, idx_p0, idx_p1, idx_n0, idx_n1,
                    rows_v0, rows_v1, rows_p0, rows_p1, rows_n0, rows_n1,
                    sp, sn, sem0, sem1):
    wid = lax.axis_index("s") * NC + lax.axis_index("c")
    idx_c = (idx_c0, idx_c1)
    idx_p = (idx_p0, idx_p1)
    idx_n = (idx_n0, idx_n1)
    rows_v = (rows_v0, rows_v1)
    rows_p = (rows_p0, rows_p1)
    rows_n = (rows_n0, rows_n1)
    sems = (sem0, sem1)
    iota16 = lax.broadcasted_iota(jnp.int32, (16,), 0)

    def issue(c, s):
        base = wid * ROWS_W + c * CHUNK
        for g in range(CHUNK // 16):
            idx_c[s][pl.ds(g * 16, 16)] = base + g * 16 + iota16
        pltpu.sync_copy(pos_hbm.at[pl.ds(base, CHUNK)], idx_p[s])
        pltpu.sync_copy(neg_hbm.at[pl.ds(base * NEG, NEG_CH)], idx_n[s])
        cps = [pltpu.async_copy(v_hbm.at[idx_c[s]], rows_v[s], sems[s]),
               pltpu.async_copy(out_hbm.at[idx_p[s]], rows_p[s], sems[s])]
        for k in range(NEG_GATHERS):
            cps.append(pltpu.async_copy(out_hbm.at[idx_n[s].at[pl.ds(k * 128, 128)]],
                                        rows_n[s].at[pl.ds(k * 128, 128)],
                                        sems[s]))
        return cps

    def compute(c, s):
        for g in range(CHUNK // 16):
            r_idx = g * 16 + iota16
            p_idx = [(g * 16 + iota16) * NEG + j for j in range(NEG)]
            zeros = jnp.zeros((16,), jnp.float32)

            def body(d, accs):
                col = jnp.broadcast_to(d, (16,))
                vv = plsc.load_gather(rows_v[s], [r_idx, col])
                up = plsc.load_gather(rows_p[s], [r_idx, col])
                new = [accs[0] + vv * up]
                for j in range(NEG):
                    un = plsc.load_gather(rows_n[s], [p_idx[j], col])
                    new.append(accs[1 + j] + vv * un)
                return tuple(new)

            accs = lax.fori_loop(0, EMB, body, (zeros,) * (1 + NEG))
            sp[pl.ds(c * CHUNK + g * 16, 16)] = accs[0]
            for j in range(NEG):
                plsc.store_scatter(
                    sn, [(c * CHUNK + g * 16 + iota16) * NEG + j], accs[1 + j])

    cps = issue(0, 0)
    for c in range(NCHUNK):
        s = c % 2
        nxt = issue(c + 1, 1 - s) if c + 1 < NCHUNK else None
        for cp in cps:
            cp.wait()
        compute(c, s)
        cps = nxt

    pltpu.sync_copy(sp, sp_out.at[pl.ds(wid * ROWS_W, ROWS_W)])
    pltpu.sync_copy(sn, sn_out.at[pl.ds(wid * ROWS_W * NEG, ROWS_W * NEG)])


def _loss_body(sp_ref, sn_ref, out_ref):
    ps = sp_ref[...]
    ns = sn_ref[...]
    pls = jnp.minimum(ps, 0.0) - jnp.log1p(jnp.exp(-jnp.abs(ps)))
    nls = jnp.minimum(-ns, 0.0) - jnp.log1p(jnp.exp(-jnp.abs(ns)))
    out_ref[0, 0] = -(jnp.sum(pls) / BATCH) - (jnp.sum(nls) / (BATCH * NEG))


@jax.jit
def kernel(in_embed, out_embed, center, pos, neg):
    center = center.astype(jnp.int32)
    pos = pos.astype(jnp.int32)
    neg_flat = jnp.reshape(neg.astype(jnp.int32), (BATCH * NEG,))
    v = jnp.take(in_embed, center, axis=0)

    mesh = plsc.VectorSubcoreMesh(core_axis_name="c", subcore_axis_name="s")
    sc_scores = functools.partial(
        pl.kernel,
        mesh=mesh,
        compiler_params=pltpu.CompilerParams(
            needs_layout_passes=False, use_tc_tiling_on_sc=False),
        out_type=[jax.ShapeDtypeStruct((BATCH,), jnp.float32),
                  jax.ShapeDtypeStruct((BATCH * NEG,), jnp.float32)],
        scratch_types=[
            pltpu.VMEM((CHUNK,), jnp.int32), pltpu.VMEM((CHUNK,), jnp.int32),
            pltpu.VMEM((CHUNK,), jnp.int32), pltpu.VMEM((CHUNK,), jnp.int32),
            pltpu.VMEM((NEG_CH,), jnp.int32),
            pltpu.VMEM((NEG_CH,), jnp.int32),
            pltpu.VMEM((CHUNK, EMB), jnp.float32),
            pltpu.VMEM((CHUNK, EMB), jnp.float32),
            pltpu.VMEM((CHUNK, EMB), jnp.float32),
            pltpu.VMEM((CHUNK, EMB), jnp.float32),
            pltpu.VMEM((NEG_CH, EMB), jnp.float32),
            pltpu.VMEM((NEG_CH, EMB), jnp.float32),
            pltpu.VMEM((ROWS_W,), jnp.float32),
            pltpu.VMEM((ROWS_W * NEG,), jnp.float32),
            pltpu.SemaphoreType.DMA,
            pltpu.SemaphoreType.DMA,
        ],
    )(_sc_scores_body)
    sp, sn = sc_scores(pos, neg_flat, v, out_embed)

    loss = pl.pallas_call(
        _loss_body,
        out_shape=jax.ShapeDtypeStruct((1, 1), jnp.float32),
        out_specs=pl.BlockSpec(memory_space=pltpu.SMEM),
    )(jnp.reshape(sp, (BATCH // 128, 128)),
      jnp.reshape(sn, (BATCH * NEG // 128, 128)))
    return loss[0, 0]
